# baseline (speedup 1.0000x reference)
import functools
import math

import jax
import jax.numpy as jnp
from jax import lax
from jax.experimental import pallas as pl
from jax.experimental.pallas import tpu as pltpu

HEADS = 8
SAMPLES = 4
SLABS = 2
HI = lax.Precision.HIGHEST


def _layernorm(x, g=None, b=None, eps=1e-5):
    mu = jnp.mean(x, axis=-1, keepdims=True)
    var = jnp.mean((x - mu) ** 2, axis=-1, keepdims=True)
    y = (x - mu) / jnp.sqrt(var + eps)
    if g is not None:
        y = y * g + b
    return y


def _slab(xx, wq_ref, bq_ref, wk_ref, bk_ref, wv_ref, bv_ref,
          wo_ref, bo_ref, ln_g_ref, ln_b_ref,
          wf1_ref, bf1_ref, wf2_ref, bf2_ref,
          wp_ref, la_ref, *, n, f, s):
    h = HEADS
    d = f // h
    f32 = jnp.float32
    i32 = jnp.int32
    bfr = lambda a: a.astype(jnp.bfloat16).astype(f32)

    q = jnp.dot(xx, wq_ref[...], preferred_element_type=f32) + bq_ref[...]
    k = jnp.dot(xx, wk_ref[...], preferred_element_type=f32) + bk_ref[...]
    v = jnp.dot(xx, wv_ref[...], preferred_element_type=f32) + bv_ref[...]

    nk = la_ref.shape[0]
    land = (lax.broadcasted_iota(i32, (nk, n), 1) == la_ref[...]).astype(f32)
    kland = jnp.dot(land, k, preferred_element_type=f32, precision=HI)
    wp_col = bfr(wp_ref[...])[:, None, :]

    ia_lt_ib = (lax.broadcasted_iota(i32, (n, n), 0)
                < lax.broadcasted_iota(i32, (n, n), 1))
    sel_iota = lax.broadcasted_iota(i32, (s, n), 0)
    ridx = sel_iota
    den = jnp.float32(math.sqrt(d))

    heads = []
    for hh in range(h):
        sl = slice(hh * d, (hh + 1) * d)
        qh, kh, vh = q[:, sl], k[:, sl], v[:, sl]
        qks = jnp.sum(qh[None, :, :] * kland[:, sl][:, None, :], axis=2)
        m_row = jnp.sum(bfr(qks) * wp_col[:, :, 0], axis=0, keepdims=True)
        m_col = jnp.transpose(m_row)
        before = jnp.where(
            (m_col > m_row) | ((m_col == m_row) & ia_lt_ib), 1.0, 0.0)
        rank_row = jnp.sum(before, axis=0, keepdims=True).astype(i32)
        sel = (sel_iota == rank_row).astype(f32)
        q_red = jnp.dot(sel, qh, preferred_element_type=f32, precision=HI)
        qk = lax.dot_general(q_red, kh, (((1,), (1,)), ((), ())),
                             preferred_element_type=f32) / den
        mx = jnp.max(qk, axis=1, keepdims=True)
        ex = jnp.exp(qk - mx)
        attn = ex / jnp.sum(ex, axis=1, keepdims=True)
        colmax = jnp.max(attn, axis=0, keepdims=True)
        cand = jnp.where(attn == colmax, ridx, n + s)
        cp = jnp.min(cand, axis=0, keepdims=True)
        oh = (ridx == cp).astype(f32)
        av = jnp.dot(attn, vh, preferred_element_type=f32)
        vh_out = lax.dot_general(oh, av, (((0,), (0,)), ((), ())),
                                 preferred_element_type=f32, precision=HI)
        heads.append(vh_out)

    val = jnp.concatenate(heads, axis=1)
    out = jnp.dot(val, wo_ref[...], preferred_element_type=f32) + bo_ref[...]
    out = _layernorm(out, ln_g_ref[...], ln_b_ref[...])
    y1 = jnp.maximum(jnp.dot(out, wf1_ref[...], preferred_element_type=f32)
                     + bf1_ref[...], 0.0)
    y2 = jnp.dot(y1, wf2_ref[...], preferred_element_type=f32) + bf2_ref[...]
    return _layernorm(y2 + out)


def _body(x_ref, spa_vec_ref, spa_val_ref, tem_vec_ref, tem_val_ref,
          wq_ref, bq_ref, wk_ref, bk_ref, wv_ref, bv_ref,
          wo_ref, bo_ref, ln_g_ref, ln_b_ref,
          wf1_ref, bf1_ref, wf2_ref, bf2_ref,
          wp_ref, la_ref, o_ref, spa_ref, tem_ref, *, n, f, s):
    f32 = jnp.float32
    bfr = lambda a: a.astype(jnp.bfloat16).astype(f32)

    @pl.when(pl.program_id(0) == 0)
    def _():
        spa_ref[...] = bfr(spa_vec_ref[...]) * bfr(spa_val_ref[...])
        tem_ref[...] = bfr(tem_vec_ref[...]) * bfr(tem_val_ref[...])

    spa_pos = spa_ref[...]
    tem_pos = tem_ref[...]
    rest = (wq_ref, bq_ref, wk_ref, bk_ref, wv_ref, bv_ref,
            wo_ref, bo_ref, ln_g_ref, ln_b_ref,
            wf1_ref, bf1_ref, wf2_ref, bf2_ref, wp_ref, la_ref)
    for sb in range(SLABS):
        o_ref[sb] = _slab((x_ref[sb] + spa_pos) + tem_pos, *rest,
                          n=n, f=f, s=s)


def kernel(x, spa_eigvalue, spa_eigvec, tem_eigvalue, tem_eigvec,
           Wq, bq, Wk, bk, Wv, bv, Wo, bo, ln_g, ln_b,
           Wf1, bf1, Wf2, bf2, Wp, bp, localadj):
    b0, t, n, f = x.shape
    s = int(SAMPLES * math.log(n, 2))
    xr = x.reshape(b0 * t, n, f)

    row = lambda a: a.reshape(1, f)
    full = lambda a: pl.BlockSpec(a.shape, lambda i: (0,) * a.ndim)

    operands = (
        xr, spa_eigvec, row(spa_eigvalue), tem_eigvec, row(tem_eigvalue),
        Wq.T, row(bq), Wk.T, row(bk), Wv.T, row(bv),
        Wo.T, row(bo), row(ln_g), row(ln_b),
        Wf1.T, row(bf1), Wf2.T, row(bf2),
        Wp.reshape(-1, 1), localadj.reshape(-1, 1),
    )
    in_specs = [pl.BlockSpec((SLABS, n, f), lambda i: (i, 0, 0))]
    in_specs += [full(a) for a in operands[1:]]

    out = pl.pallas_call(
        functools.partial(_body, n=n, f=f, s=s),
        grid=(b0 * t // SLABS,),
        in_specs=in_specs,
        out_specs=pl.BlockSpec((SLABS, n, f), lambda i: (i, 0, 0)),
        out_shape=jax.ShapeDtypeStruct((b0 * t, n, f), jnp.float32),
        scratch_shapes=[pltpu.VMEM((n, f), jnp.float32),
                        pltpu.VMEM((n, f), jnp.float32)],
    )(*operands)
    return out.reshape(b0, t, n, f)


# speedup vs baseline: 1.0945x; 1.0945x over previous
import functools
import math

import jax
import jax.numpy as jnp
from jax import lax
from jax.experimental import pallas as pl

HEADS = 8
SAMPLES = 4
HI = lax.Precision.HIGHEST


def _layernorm(x, g=None, b=None, eps=1e-5):
    mu = jnp.mean(x, axis=-1, keepdims=True)
    var = jnp.mean((x - mu) ** 2, axis=-1, keepdims=True)
    y = (x - mu) / jnp.sqrt(var + eps)
    if g is not None:
        y = y * g + b
    return y


def _body(x_ref, spa_vec_ref, spa_val_ref, tem_vec_ref, tem_val_ref,
          wq_ref, bq_ref, wk_ref, bk_ref, wv_ref, bv_ref,
          wo_ref, bo_ref, ln_g_ref, ln_b_ref,
          wf1_ref, bf1_ref, wf2_ref, bf2_ref,
          wp_ref, la_ref, o_ref, *, n, f, s):
    h = HEADS
    d = f // h
    f32 = jnp.float32
    i32 = jnp.int32
    bfr = lambda a: a.astype(jnp.bfloat16).astype(f32)

    xb = x_ref[0]
    spa_pos = bfr(spa_vec_ref[...]) * bfr(spa_val_ref[...])
    tem_pos = bfr(tem_vec_ref[...]) * bfr(tem_val_ref[...])
    xx = (xb + spa_pos) + tem_pos

    q = jnp.dot(xx, wq_ref[...], preferred_element_type=f32) + bq_ref[...]
    k = jnp.dot(xx, wk_ref[...], preferred_element_type=f32) + bk_ref[...]
    v = jnp.dot(xx, wv_ref[...], preferred_element_type=f32) + bv_ref[...]

    nk = la_ref.shape[0]
    land = (lax.broadcasted_iota(i32, (nk, n), 1) == la_ref[...]).astype(f32)
    kland = jnp.dot(land, k, preferred_element_type=f32, precision=HI)

    jl_lt_il = (lax.broadcasted_iota(i32, (n, n), 1)
                < lax.broadcasted_iota(i32, (n, n), 0))
    rank_iota = lax.broadcasted_iota(i32, (n, s), 1)
    ridx = lax.broadcasted_iota(i32, (s, n), 0)
    den = jnp.float32(math.sqrt(d))

    heads = []
    for hh in range(h):
        sl = slice(hh * d, (hh + 1) * d)
        qh, kh, vh = q[:, sl], k[:, sl], v[:, sl]
        m_col = None
        for j in range(nk):
            qks_j = jnp.sum(qh * kland[j:j + 1, sl], axis=1, keepdims=True)
            term = bfr(qks_j) * bfr(wp_ref[j:j + 1, 0:1])
            m_col = term if m_col is None else m_col + term
        m_row = jnp.transpose(m_col)
        before = jnp.where(
            (m_row > m_col) | ((m_row == m_col) & jl_lt_il), 1.0, 0.0)
        rank = jnp.sum(before, axis=1, keepdims=True).astype(i32)
        sel_t = (rank == rank_iota).astype(f32)
        q_red = lax.dot_general(sel_t, qh, (((0,), (0,)), ((), ())),
                                preferred_element_type=f32, precision=HI)
        qk = lax.dot_general(q_red, kh, (((1,), (1,)), ((), ())),
                             preferred_element_type=f32) / den
        mx = jnp.max(qk, axis=1, keepdims=True)
        ex = jnp.exp(qk - mx)
        attn = ex / jnp.sum(ex, axis=1, keepdims=True)
        colmax = jnp.max(attn, axis=0, keepdims=True)
        cand = jnp.where(attn == colmax, ridx, n + s)
        cp = jnp.min(cand, axis=0, keepdims=True)
        oh = (ridx == cp).astype(f32)
        av = jnp.dot(attn, vh, preferred_element_type=f32)
        vh_out = lax.dot_general(oh, av, (((0,), (0,)), ((), ())),
                                 preferred_element_type=f32, precision=HI)
        heads.append(vh_out)

    val = jnp.concatenate(heads, axis=1)
    out = jnp.dot(val, wo_ref[...], preferred_element_type=f32) + bo_ref[...]
    out = _layernorm(out, ln_g_ref[...], ln_b_ref[...])
    y1 = jnp.maximum(jnp.dot(out, wf1_ref[...], preferred_element_type=f32)
                     + bf1_ref[...], 0.0)
    y2 = jnp.dot(y1, wf2_ref[...], preferred_element_type=f32) + bf2_ref[...]
    o_ref[0] = _layernorm(y2 + out)


def kernel(x, spa_eigvalue, spa_eigvec, tem_eigvalue, tem_eigvec,
           Wq, bq, Wk, bk, Wv, bv, Wo, bo, ln_g, ln_b,
           Wf1, bf1, Wf2, bf2, Wp, bp, localadj):
    b0, t, n, f = x.shape
    s = int(SAMPLES * math.log(n, 2))
    xr = x.reshape(b0 * t, n, f)

    row = lambda a: a.reshape(1, f)
    full = lambda a: pl.BlockSpec(a.shape, lambda i: (0,) * a.ndim)

    operands = (
        xr, spa_eigvec, row(spa_eigvalue), tem_eigvec, row(tem_eigvalue),
        Wq.T, row(bq), Wk.T, row(bk), Wv.T, row(bv),
        Wo.T, row(bo), row(ln_g), row(ln_b),
        Wf1.T, row(bf1), Wf2.T, row(bf2),
        Wp.reshape(-1, 1), localadj.reshape(-1, 1),
    )
    in_specs = [pl.BlockSpec((1, n, f), lambda i: (i, 0, 0))]
    in_specs += [full(a) for a in operands[1:]]

    out = pl.pallas_call(
        functools.partial(_body, n=n, f=f, s=s),
        grid=(b0 * t,),
        in_specs=in_specs,
        out_specs=pl.BlockSpec((1, n, f), lambda i: (i, 0, 0)),
        out_shape=jax.ShapeDtypeStruct((b0 * t, n, f), jnp.float32),
    )(*operands)
    return out.reshape(b0, t, n, f)
